# baseline (speedup 1.0000x reference)
import functools

import jax
import jax.numpy as jnp
from jax import lax
from jax.experimental import pallas as pl
from jax.experimental.pallas import tpu as pltpu
from jax.experimental.pallas import tpu_sc as plsc

_B = 16384
_D = 64
_REL = 5
_L = 16
_NC = 2
_NS = 16
_NW = _NC * _NS
_BPW = _B // _NW
_CH = 128
_NCH = _BPW // _CH


def _vsqrt(x):
    i = plsc.bitcast(x, jnp.int32)
    i = jnp.int32(0x5F3759DF) - lax.shift_right_logical(i, 1)
    y = plsc.bitcast(i, jnp.float32)
    for _ in range(3):
        y = y * (1.5 - 0.5 * x * y * y)
    return x * y


def _score_body(data_hbm, nid_hbm, ent_hbm, rel_hbm, out_hbm,
                data_v, hi_v, ti_v, hm_v, tm_v, hrows_v, trows_v, rel_v,
                out_v, sem0, sem1):
    wid = lax.axis_index("s") * _NC + lax.axis_index("c")
    base = wid * _BPW
    pltpu.sync_copy(rel_hbm.at[_REL], rel_v)
    zeros = jnp.zeros((_L,), jnp.int32)
    ones = jnp.ones((_L,), jnp.int32)
    for c in range(_NCH):
        off = base + c * _CH
        pltpu.sync_copy(data_hbm.at[pl.ds(off, _CH)], data_v)
        for g in range(_CH // _L):
            lanes = lax.iota(jnp.int32, _L) + g * _L
            hi_v[pl.ds(g * _L, _L)] = plsc.load_gather(data_v, [lanes, zeros])
            ti_v[pl.ds(g * _L, _L)] = plsc.load_gather(data_v, [lanes, ones])
        cp_h = pltpu.async_copy(nid_hbm.at[hi_v], hm_v, sem0)
        cp_t = pltpu.async_copy(nid_hbm.at[ti_v], tm_v, sem1)
        cp_h.wait()
        cp_t.wait()
        cp_hr = pltpu.async_copy(ent_hbm.at[hm_v], hrows_v, sem0)
        cp_tr = pltpu.async_copy(ent_hbm.at[tm_v], trows_v, sem1)
        cp_hr.wait()
        cp_tr.wait()

        def group_body(g, carry):
            rows = lax.iota(jnp.int32, _L) + g * _L

            def dim_body(d, acc):
                dv = jnp.full((_L,), d, dtype=jnp.int32)
                hd = plsc.load_gather(hrows_v, [rows, dv])
                td = plsc.load_gather(trows_v, [rows, dv])
                rd = plsc.load_gather(rel_v, [dv])
                diff = hd - td + rd
                return acc + diff * diff

            acc = lax.fori_loop(0, _D, dim_body,
                                jnp.zeros((_L,), jnp.float32))
            s = _vsqrt(acc + 1e-12)
            out_v[pl.ds(c * _CH + g * _L, _L)] = s
            return carry

        lax.fori_loop(0, _CH // _L, group_body, 0)
    pltpu.sync_copy(out_v, out_hbm.at[pl.ds(base, _BPW)])


_mesh = plsc.VectorSubcoreMesh(core_axis_name="c", subcore_axis_name="s",
                               num_cores=_NC, num_subcores=_NS)

_sc_score = pl.kernel(
    _score_body,
    out_type=jax.ShapeDtypeStruct((_B,), jnp.float32),
    mesh=_mesh,
    compiler_params=pltpu.CompilerParams(use_tc_tiling_on_sc=False,
                                         needs_layout_passes=False),
    scratch_types=[
        pltpu.VMEM((_CH, 2), jnp.int32),
        pltpu.VMEM((_CH,), jnp.int32),
        pltpu.VMEM((_CH,), jnp.int32),
        pltpu.VMEM((_CH,), jnp.int32),
        pltpu.VMEM((_CH,), jnp.int32),
        pltpu.VMEM((_CH, _D), jnp.float32),
        pltpu.VMEM((_CH, _D), jnp.float32),
        pltpu.VMEM((_D,), jnp.float32),
        pltpu.VMEM((_BPW,), jnp.float32),
        pltpu.SemaphoreType.DMA,
        pltpu.SemaphoreType.DMA,
    ],
)


@jax.jit
def kernel(data, node_ids, entity_emb, relation_emb):
    out = _sc_score(data, node_ids, entity_emb, relation_emb)
    return out[:, None]


# speedup vs baseline: 1.4488x; 1.4488x over previous
import jax
import jax.numpy as jnp
from jax import lax
from jax.experimental import pallas as pl
from jax.experimental.pallas import tpu as pltpu
from jax.experimental.pallas import tpu_sc as plsc

_B = 16384
_D = 64
_REL = 5
_L = 16
_NC = 2
_NS = 16
_NW = _NC * _NS
_BPW = _B // _NW
_CH = 128
_NCH = _BPW // _CH
_NG = _CH // _L


def _vsqrt(x):
    i = plsc.bitcast(x, jnp.int32)
    i = jnp.int32(0x5F3759DF) - lax.shift_right_logical(i, 1)
    y = plsc.bitcast(i, jnp.float32)
    for _ in range(3):
        y = y * (1.5 - 0.5 * x * y * y)
    return x * y


def _score_body(hidx_hbm, tidx_hbm, nid_hbm, ent_hbm, rel_hbm, out_hbm,
                hi_v, ti_v, hm_v, tm_v,
                hblk_v, tblk_v, rel_v, out_v, sem0, sem1):
    wid = lax.axis_index("s") * _NC + lax.axis_index("c")
    base = wid * _BPW
    pltpu.sync_copy(rel_hbm, rel_v)
    lanes = lax.iota(jnp.int32, _L)
    def chunk_body(c, carry0):
        off = base + c * _CH
        pltpu.sync_copy(hidx_hbm.at[pl.ds(off, _CH)], hi_v)
        pltpu.sync_copy(tidx_hbm.at[pl.ds(off, _CH)], ti_v)
        cp_h = pltpu.async_copy(nid_hbm.at[hi_v], hm_v, sem0)
        cp_t = pltpu.async_copy(nid_hbm.at[ti_v], tm_v, sem1)
        cp_h.wait()
        cp_t.wait()

        def group_body(g, carry1):
            hm16 = hm_v[pl.ds(g * _L, _L)]
            tm16 = tm_v[pl.ds(g * _L, _L)]
            for j in range(_L):
                bh = pl.multiple_of((hm16[j] // 8) * 8, 8)
                bt = pl.multiple_of((tm16[j] // 8) * 8, 8)
                pltpu.async_copy(ent_hbm.at[pl.ds(bh, 8)],
                                 hblk_v.at[pl.ds(j * 8, 8)], sem0)
                pltpu.async_copy(ent_hbm.at[pl.ds(bt, 8)],
                                 tblk_v.at[pl.ds(j * 8, 8)], sem1)
            pltpu.make_async_copy(ent_hbm.at[pl.ds(0, _L * 8)], hblk_v,
                                  sem0).wait()
            pltpu.make_async_copy(ent_hbm.at[pl.ds(0, _L * 8)], tblk_v,
                                  sem1).wait()
            hrow = lanes * 8 + (hm16 & 7)
            trow = lanes * 8 + (tm16 & 7)

            def dim_body(d, acc):
                dv = jnp.full((_L,), d, jnp.int32)
                hd = plsc.load_gather(hblk_v, [hrow, dv])
                td = plsc.load_gather(tblk_v, [trow, dv])
                rd = plsc.load_gather(rel_v, [dv])
                diff = hd - td + rd
                return acc + diff * diff

            acc = lax.fori_loop(0, _D, dim_body,
                                jnp.zeros((_L,), jnp.float32))
            out_v[pl.ds(c * _CH + g * _L, _L)] = _vsqrt(acc + 1e-12)
            return carry1

        lax.fori_loop(0, _NG, group_body, 0)
        return carry0

    lax.fori_loop(0, _NCH, chunk_body, 0)
    pltpu.sync_copy(out_v, out_hbm.at[pl.ds(base, _BPW)])


_mesh = plsc.VectorSubcoreMesh(core_axis_name="c", subcore_axis_name="s",
                               num_cores=_NC, num_subcores=_NS)

_sc_score = pl.kernel(
    _score_body,
    out_type=jax.ShapeDtypeStruct((_B,), jnp.float32),
    mesh=_mesh,
    compiler_params=pltpu.CompilerParams(needs_layout_passes=False),
    scratch_types=[
        pltpu.VMEM((_CH,), jnp.int32),
        pltpu.VMEM((_CH,), jnp.int32),
        pltpu.VMEM((_CH,), jnp.int32),
        pltpu.VMEM((_CH,), jnp.int32),
        pltpu.VMEM((_L * 8, _D), jnp.float32),
        pltpu.VMEM((_L * 8, _D), jnp.float32),
        pltpu.VMEM((_D,), jnp.float32),
        pltpu.VMEM((_BPW,), jnp.float32),
        pltpu.SemaphoreType.DMA,
        pltpu.SemaphoreType.DMA,
    ],
)


@jax.jit
def kernel(data, node_ids, entity_emb, relation_emb):
    hidx = data[:, 0]
    tidx = data[:, 1]
    rel_row = relation_emb[_REL]
    out = _sc_score(hidx, tidx, node_ids, entity_emb, rel_row)
    return out[:, None]
